# Initial kernel scaffold; baseline (speedup 1.0000x reference)
import jax
import jax.numpy as jnp
from jax.experimental import pallas as pl


def kernel(x, edge_index, W1, a_src1, a_dst1, b1, W2, a_src2, a_dst2, b2):
    raise NotImplementedError("write your pallas kernel here")


# speedup vs baseline: 44.0712x; 44.0712x over previous
import functools

import jax
import jax.numpy as jnp
from jax import lax
from jax.experimental import pallas as pl
from jax.experimental.pallas import tpu as pltpu
from jax.experimental.pallas import tpu_sc as plsc

N = 10000
EDGES = 320000
D_IN = 128
WIDTH = 80
R = 10240
NW = 32
CH = 128
NCHUNK = 79
EPW = NCHUNK * CH
EP = NW * EPW
RPW = R // 16
TBLK = 256
FBLK = 128


def _edge_body(hs_hbm, adp_hbm, src_hbm, dst_hbm, out_hbm,
               srci, dsti, srows, adrows, orows, stage, acc,
               sem1, sem2):
    c = lax.axis_index("c")
    s = lax.axis_index("s")
    wid = s * 2 + c
    iota = lax.iota(jnp.int32, 16)
    hi = lax.shift_right_logical(iota, 3)
    z16 = jnp.zeros((16,), jnp.float32)

    def vperm(v, idx):
        return v.at[idx].get(mode="promise_in_bounds")

    def zrow(i, carry):
        for j in range(WIDTH // 16):
            stage[i, pl.ds(16 * j, 16)] = z16
        return carry
    lax.fori_loop(0, CH, zrow, 0)
    for t in range(RPW // CH):
        pltpu.sync_copy(stage, acc.at[pl.ds(s * RPW + t * CH, CH)])
    plsc.subcore_barrier()

    def chunk(k, carry):
        e0 = wid * EPW + k * CH
        pltpu.sync_copy(src_hbm.at[pl.ds(e0, CH)], srci)
        pltpu.sync_copy(dst_hbm.at[pl.ds(e0, CH)], dsti)
        d1 = pltpu.async_copy(hs_hbm.at[srci], srows, sem1)
        d2 = pltpu.async_copy(adp_hbm.at[dsti], adrows, sem2)
        d1.wait()
        d2.wait()

        def edge_row(e, carry):
            a_s = srows[e, pl.ds(64, 16)]
            a_d = adrows[e]
            ev = a_s + a_d
            ev = jnp.where(ev >= 0.0, ev, 0.2 * ev)
            ee = jnp.exp(ev)
            for j in range(4):
                h_v = srows[e, pl.ds(16 * j, 16)]
                orows[e, pl.ds(16 * j, 16)] = h_v * vperm(ee, 2 * j + hi)
            orows[e, pl.ds(64, 16)] = jnp.where(iota < 8, ee, 0.0)
            return carry
        lax.fori_loop(0, CH, edge_row, 0)

        pltpu.sync_copy(orows, acc.at[dsti], add=True)
        return carry
    lax.fori_loop(0, NCHUNK, chunk, 0)

    plsc.subcore_barrier()
    for t in range(RPW // CH):
        r0 = s * RPW + t * CH
        pltpu.sync_copy(acc.at[pl.ds(r0, CH)], stage)
        pltpu.sync_copy(stage, out_hbm.at[pl.ds(c * R + r0, CH)])


def _edge_pass(hs, adp, srcp, dstp):
    mesh = plsc.VectorSubcoreMesh(core_axis_name="c", subcore_axis_name="s",
                                  num_cores=2, num_subcores=16)
    run = pl.kernel(
        _edge_body,
        out_type=jax.ShapeDtypeStruct((2 * R, WIDTH), jnp.float32),
        mesh=mesh,
        scratch_types=[
            pltpu.VMEM((CH,), jnp.int32),
            pltpu.VMEM((CH,), jnp.int32),
            pltpu.VMEM((CH, WIDTH), jnp.float32),
            pltpu.VMEM((CH, 16), jnp.float32),
            pltpu.VMEM((CH, WIDTH), jnp.float32),
            pltpu.VMEM((CH, WIDTH), jnp.float32),
            pltpu.VMEM_SHARED((R, WIDTH), jnp.float32),
            pltpu.SemaphoreType.DMA,
            pltpu.SemaphoreType.DMA,
        ],
        compiler_params=pltpu.CompilerParams(use_tc_tiling_on_sc=False),
    )
    return run(hs, adp, srcp, dstp)


def _tc1_body(x_ref, w_ref, as_ref, ad_ref, hs_ref, adp_ref):
    h = jnp.dot(x_ref[...], w_ref[...], preferred_element_type=jnp.float32)
    a_s = jnp.dot(h, as_ref[...], preferred_element_type=jnp.float32)
    a_d = jnp.dot(h, ad_ref[...], preferred_element_type=jnp.float32)
    pad = jnp.zeros((h.shape[0], 8), jnp.float32)
    hs_ref[...] = jnp.concatenate([h, a_s, pad], axis=1)
    adp_ref[...] = jnp.concatenate([a_d, pad], axis=1)


def _node_tables_l1(xp, W1, As1, Ad1):
    return pl.pallas_call(
        _tc1_body,
        grid=(R // TBLK,),
        in_specs=[
            pl.BlockSpec((TBLK, D_IN), lambda i: (i, 0)),
            pl.BlockSpec((D_IN, 64), lambda i: (0, 0)),
            pl.BlockSpec((64, 8), lambda i: (0, 0)),
            pl.BlockSpec((64, 8), lambda i: (0, 0)),
        ],
        out_specs=[pl.BlockSpec((TBLK, WIDTH), lambda i: (i, 0)),
                   pl.BlockSpec((TBLK, 16), lambda i: (i, 0))],
        out_shape=[jax.ShapeDtypeStruct((R, WIDTH), jnp.float32),
                   jax.ShapeDtypeStruct((R, 16), jnp.float32)],
    )(xp, W1, As1, Ad1)


def _tc2_body(acc_ref, b_ref, w_ref, as_ref, ad_ref, er_ref, hs_ref, adp_ref):
    a = acc_ref[0] + acc_ref[1]
    msg = a[:, :64]
    den = jnp.dot(a[:, 64:72], er_ref[...], preferred_element_type=jnp.float32)
    z = msg / (den + 1e-16) + b_ref[...]
    z = jnp.where(z > 0.0, z, jnp.exp(z) - 1.0)
    rows = (pl.program_id(0) * TBLK
            + lax.broadcasted_iota(jnp.int32, (TBLK, 1), 0))
    z = jnp.where(rows < N, z, 0.0)
    h2 = jnp.dot(z, w_ref[...], preferred_element_type=jnp.float32)
    a_s = jnp.dot(h2, as_ref[...], preferred_element_type=jnp.float32)
    a_d = jnp.dot(h2, ad_ref[...], preferred_element_type=jnp.float32)
    pad = jnp.zeros((h2.shape[0], 8), jnp.float32)
    hs_ref[...] = jnp.concatenate([h2, a_s, pad], axis=1)
    adp_ref[...] = jnp.concatenate([a_d, pad], axis=1)


def _node_tables_l2(acc1, b1r, W2, As2, Ad2, Erep):
    return pl.pallas_call(
        _tc2_body,
        grid=(R // TBLK,),
        in_specs=[
            pl.BlockSpec((2, TBLK, WIDTH), lambda i: (0, i, 0)),
            pl.BlockSpec((1, 64), lambda i: (0, 0)),
            pl.BlockSpec((64, 64), lambda i: (0, 0)),
            pl.BlockSpec((64, 8), lambda i: (0, 0)),
            pl.BlockSpec((64, 8), lambda i: (0, 0)),
            pl.BlockSpec((8, 64), lambda i: (0, 0)),
        ],
        out_specs=[pl.BlockSpec((TBLK, WIDTH), lambda i: (i, 0)),
                   pl.BlockSpec((TBLK, 16), lambda i: (i, 0))],
        out_shape=[jax.ShapeDtypeStruct((R, WIDTH), jnp.float32),
                   jax.ShapeDtypeStruct((R, 16), jnp.float32)],
    )(acc1, b1r, W2, As2, Ad2, Erep)


def _tc3_body(acc_ref, b_ref, o_ref):
    a = acc_ref[0] + acc_ref[1]
    o = a[:, :64] / (a[:, 64:65] + 1e-16) + b_ref[...]
    o = o - jnp.max(o, axis=1, keepdims=True)
    o_ref[...] = o - jnp.log(jnp.sum(jnp.exp(o), axis=1, keepdims=True))


def _final(acc2, b2r):
    return pl.pallas_call(
        _tc3_body,
        grid=(pl.cdiv(N, FBLK),),
        in_specs=[
            pl.BlockSpec((2, FBLK, WIDTH), lambda i: (0, i, 0)),
            pl.BlockSpec((1, 64), lambda i: (0, 0)),
        ],
        out_specs=pl.BlockSpec((FBLK, 64), lambda i: (i, 0)),
        out_shape=jax.ShapeDtypeStruct((N, 64), jnp.float32),
    )(acc2, b2r)


def kernel(x, edge_index, W1, a_src1, a_dst1, b1, W2, a_src2, a_dst2, b2):
    f32 = jnp.float32
    xp = jnp.pad(x.astype(f32), ((0, R - N), (0, 0)))
    ei = edge_index.astype(jnp.int32)
    srcp = jnp.pad(ei[0], (0, EP - EDGES))
    dstp = jnp.pad(ei[1], (0, EP - EDGES), constant_values=N)

    eye8 = jnp.eye(8, dtype=f32)
    As1 = (eye8[:, None, :] * a_src1[:, :, None]).reshape(64, 8)
    Ad1 = (eye8[:, None, :] * a_dst1[:, :, None]).reshape(64, 8)
    As2 = jnp.tile(a_src2.reshape(64, 1), (1, 8))
    Ad2 = jnp.tile(a_dst2.reshape(64, 1), (1, 8))
    Erep = (eye8[:, :, None] * jnp.ones((1, 1, 8), f32)).reshape(8, 64)

    hs1, adp1 = _node_tables_l1(xp, W1, As1, Ad1)
    acc1 = _edge_pass(hs1, adp1, srcp, dstp).reshape(2, R, WIDTH)
    hs2, adp2 = _node_tables_l2(acc1, b1.reshape(1, 64), W2, As2, Ad2, Erep)
    acc2 = _edge_pass(hs2, adp2, srcp, dstp).reshape(2, R, WIDTH)
    return _final(acc2, b2.reshape(1, 64))
